# baseline (speedup 1.0000x reference)
import functools

import jax
import jax.numpy as jnp
from jax import lax
from jax.experimental import pallas as pl
from jax.experimental.pallas import tpu as pltpu
from jax.experimental.pallas import tpu_sc as plsc

N_NODES = 10000
HIDDEN = 128
N_PARENTS = 20000

BN = 1000
BM = 1024
BP = 2000
M_PAD = 126976
NWORK = 32
ROWS_W = M_PAD // NWORK
CH_B = 128
NCH_B = ROWS_W // CH_B
P_PAD = 20480
PAR_W = P_PAD // NWORK
CH_D = 64
NCH_D = PAR_W // CH_D


def _expand_body(x_ref, w_ref, t_ref):
    xb = x_ref[...]
    y = jnp.dot(xb, w_ref[...], preferred_element_type=jnp.float32)
    t_ref[0] = xb
    for j in range(7):
        t_ref[1 + j] = y[:, HIDDEN * j:HIDDEN * (j + 1)]
    t_ref[8] = jnp.zeros_like(xb)


def _expand(x, wcat):
    n = x.shape[0]
    return pl.pallas_call(
        _expand_body,
        grid=(n // BN,),
        in_specs=[
            pl.BlockSpec((BN, HIDDEN), lambda i: (i, 0)),
            pl.BlockSpec((HIDDEN, 7 * HIDDEN), lambda i: (0, 0)),
        ],
        out_specs=pl.BlockSpec((9, BN, HIDDEN), lambda i: (0, i, 0)),
        out_shape=jax.ShapeDtypeStruct((9, n, HIDDEN), jnp.float32),
    )(x, wcat)


def _child_body(b_ref, tm_ref, zf_ref, sf_ref, c_ref):
    b = b_ref[...]
    e = jnp.where(b > 0, b, jnp.exp(jnp.minimum(b, 0.0)) - 1.0)
    cz = jnp.dot(e, zf_ref[...], preferred_element_type=jnp.float32)
    cs = jnp.dot(e, sf_ref[...], preferred_element_type=jnp.float32)
    t = tm_ref[...]
    c_ref[...] = jnp.where(t == 2.0, cz, jnp.where(t == 3.0, cs, b))


def _child_finish(b, tm3, zft, sft):
    m = b.shape[0]
    return pl.pallas_call(
        _child_body,
        grid=(m // BM,),
        in_specs=[
            pl.BlockSpec((BM, HIDDEN), lambda i: (i, 0)),
            pl.BlockSpec((BM, 1), lambda i: (i, 0)),
            pl.BlockSpec((HIDDEN, HIDDEN), lambda i: (0, 0)),
            pl.BlockSpec((HIDDEN, HIDDEN), lambda i: (0, 0)),
        ],
        out_specs=pl.BlockSpec((BM, HIDDEN), lambda i: (i, 0)),
        out_shape=jax.ShapeDtypeStruct((m, HIDDEN), jnp.float32),
    )(b, tm3, zft, sft)


def _parent_body(s_ref, ta_ref, pf_ref, o_ref):
    s = s_ref[...]
    sp = jnp.dot(s, pf_ref[...], preferred_element_type=jnp.float32)
    e = jnp.where(s > 0, s, jnp.exp(jnp.minimum(s, 0.0)) - 1.0)
    t = ta_ref[...]
    o_ref[...] = jnp.where(t == 0.0, s, jnp.where(t == 1.0, sp, e))


def _parent_finish(s, ta3, pft):
    p = N_PARENTS
    return pl.pallas_call(
        _parent_body,
        grid=(p // BP,),
        in_specs=[
            pl.BlockSpec((BP, HIDDEN), lambda i: (i, 0)),
            pl.BlockSpec((BP, 1), lambda i: (i, 0)),
            pl.BlockSpec((HIDDEN, HIDDEN), lambda i: (0, 0)),
        ],
        out_specs=pl.BlockSpec((BP, HIDDEN), lambda i: (i, 0)),
        out_shape=jax.ShapeDtypeStruct((p, HIDDEN), jnp.float32),
    )(s, ta3, pft)


_SC_MESH = plsc.VectorSubcoreMesh(core_axis_name="c", subcore_axis_name="s")


@functools.partial(
    pl.kernel, mesh=_SC_MESH,
    out_type=jax.ShapeDtypeStruct((M_PAD, HIDDEN), jnp.float32),
    scratch_types=[
        pltpu.VMEM((4, CH_B), jnp.int32),
        pltpu.VMEM((CH_B, HIDDEN), jnp.float32),
        pltpu.VMEM((CH_B, HIDDEN), jnp.float32),
        pltpu.VMEM((CH_B, HIDDEN), jnp.float32),
        pltpu.VMEM((CH_B, HIDDEN), jnp.float32),
        pltpu.SemaphoreType.DMA,
        pltpu.SemaphoreType.DMA,
        pltpu.SemaphoreType.DMA,
    ],
)
def _sc_gather_sum(t_hbm, g_hbm, out_hbm, idx_v, b0, b1, b2, acc,
                   sem0, sem1, sem2):
    wid = lax.axis_index("s") * 2 + lax.axis_index("c")
    wbase = wid * ROWS_W

    def chunk(ci, _):
        base = wbase + ci * CH_B
        pltpu.sync_copy(g_hbm.at[pl.ds(base, CH_B)], idx_v.at[0])
        pltpu.sync_copy(g_hbm.at[pl.ds(M_PAD + base, CH_B)], idx_v.at[1])
        pltpu.sync_copy(g_hbm.at[pl.ds(2 * M_PAD + base, CH_B)], idx_v.at[2])
        c0 = pltpu.async_copy(t_hbm.at[idx_v.at[0]], b0, sem0)
        c1 = pltpu.async_copy(t_hbm.at[idx_v.at[1]], b1, sem1)
        c2 = pltpu.async_copy(t_hbm.at[idx_v.at[2]], b2, sem2)
        c0.wait()
        c1.wait()
        c2.wait()

        def row(r, _):
            for q in range(HIDDEN // 16):
                sl = pl.ds(q * 16, 16)
                acc[r, sl] = b0[r, sl] + b1[r, sl] + b2[r, sl]
            return _

        lax.fori_loop(0, CH_B, row, None)
        pltpu.sync_copy(acc, out_hbm.at[pl.ds(base, CH_B)])
        return _

    lax.fori_loop(0, NCH_B, chunk, None)


@functools.partial(
    pl.kernel, mesh=_SC_MESH,
    out_type=jax.ShapeDtypeStruct((P_PAD, HIDDEN), jnp.float32),
    scratch_types=[
        pltpu.VMEM((8, CH_D), jnp.int32),
        pltpu.VMEM((8, CH_D, HIDDEN), jnp.float32),
        pltpu.VMEM((CH_D, HIDDEN), jnp.float32),
        pltpu.SemaphoreType.DMA,
    ],
)
def _sc_segment_sum(c_hbm, gd_hbm, out_hbm, idx_v, bufs, acc, sem):
    wid = lax.axis_index("s") * 2 + lax.axis_index("c")
    wbase = wid * PAR_W

    def chunk(ci, _):
        base = wbase + ci * CH_D
        for j in range(8):
            pltpu.sync_copy(gd_hbm.at[pl.ds(j * P_PAD + base, CH_D)],
                            idx_v.at[j])
        copies = [
            pltpu.async_copy(c_hbm.at[idx_v.at[j]], bufs.at[j], sem)
            for j in range(8)
        ]
        for c in copies:
            c.wait()

        def row(r, _):
            for q in range(HIDDEN // 16):
                sl = pl.ds(q * 16, 16)
                v = bufs[0, r, sl]
                for j in range(1, 8):
                    v = v + bufs[j, r, sl]
                acc[r, sl] = v
            return _

        lax.fori_loop(0, CH_D, row, None)
        pltpu.sync_copy(acc, out_hbm.at[pl.ds(base, CH_D)])
        return _

    lax.fori_loop(0, NCH_D, chunk, None)


def kernel(x, p_w, p_final_w, z_w, z_final_w, s_w, s_final_w,
           initial_map, order_matrix, pooling, type_mask):
    n, h = x.shape
    k, m = order_matrix.shape
    p = N_PARENTS
    im = initial_map.astype(jnp.int32)
    om = order_matrix.astype(jnp.int32)
    tm = type_mask.astype(jnp.int32)
    pool = pooling.astype(jnp.int32)

    zrow = 8 * n
    base0 = jnp.where(tm == 0, 0,
            jnp.where(tm == 1, n,
            jnp.where(tm == 2, 2 * n, 5 * n)))
    g0 = base0 + im

    def gk(kk):
        omk = om[kk]
        valid = (omk >= 0) & (tm >= 2)
        imk = im[jnp.clip(omk, 0, m - 1)]
        base = jnp.where(tm == 2, (2 + kk) * n, (5 + kk) * n)
        return jnp.where(valid, base + imk, zrow)

    pad = M_PAD - m
    zpad = jnp.full((pad,), zrow, jnp.int32)
    g0 = jnp.concatenate([g0, zpad])
    g1 = jnp.concatenate([gk(1), zpad])
    g2 = jnp.concatenate([gk(2), zpad])
    tm_pad = jnp.concatenate([tm, jnp.zeros((pad,), jnp.int32)])
    tm3 = tm_pad.astype(jnp.float32).reshape(M_PAD, 1)

    starts = jnp.searchsorted(pool, jnp.arange(p)).astype(jnp.int32)
    tm_after = tm[starts]
    cnt = jnp.where(tm_after == 0, 1, 8)
    jj = jnp.arange(8, dtype=jnp.int32)[:, None]
    gd = jnp.where(jj < cnt[None, :], starts[None, :] + jj, m)
    gd = jnp.concatenate(
        [gd, jnp.full((8, P_PAD - p), m, jnp.int32)], axis=1)
    gd = gd.reshape(8 * P_PAD)
    ta3 = tm_after.astype(jnp.float32).reshape(p, 1)

    wcat = jnp.concatenate([p_w.T] + [z_w[i].T for i in range(3)]
                           + [s_w[i].T for i in range(3)], axis=1)
    t_tab = _expand(x, wcat).reshape(9 * n, h)

    g = jnp.concatenate([g0, g1, g2])
    b = _sc_gather_sum(t_tab, g)

    c = _child_finish(b, tm3, z_final_w.T, s_final_w.T)

    s = _sc_segment_sum(c, gd)

    return _parent_finish(s, ta3, p_final_w.T)


# speedup vs baseline: 3.5803x; 3.5803x over previous
import functools

import jax
import jax.numpy as jnp
from jax import lax
from jax.experimental import pallas as pl
from jax.experimental.pallas import tpu as pltpu
from jax.experimental.pallas import tpu_sc as plsc

N_NODES = 10000
HIDDEN = 128
N_PARENTS = 20000

BN = 1000
BM = 1024
BP = 2000
M_PAD = 126976
NWORK = 32
ROWS_W = M_PAD // NWORK
CH_B = 128
NCH_B = ROWS_W // CH_B
P_PAD = 20480
PAR_W = P_PAD // NWORK
CH_D = 64
NCH_D = PAR_W // CH_D


def _expand_body(x_ref, w_ref, t_ref):
    xb = x_ref[...]
    y = jnp.dot(xb, w_ref[...], preferred_element_type=jnp.float32)
    t_ref[0] = xb
    for j in range(7):
        t_ref[1 + j] = y[:, HIDDEN * j:HIDDEN * (j + 1)]
    t_ref[8] = jnp.zeros_like(xb)


def _expand(x, wcat):
    n = x.shape[0]
    return pl.pallas_call(
        _expand_body,
        grid=(n // BN,),
        in_specs=[
            pl.BlockSpec((BN, HIDDEN), lambda i: (i, 0)),
            pl.BlockSpec((HIDDEN, 7 * HIDDEN), lambda i: (0, 0)),
        ],
        out_specs=pl.BlockSpec((9, BN, HIDDEN), lambda i: (0, i, 0)),
        out_shape=jax.ShapeDtypeStruct((9, n, HIDDEN), jnp.float32),
    )(x, wcat)


def _child_body(b_ref, tm_ref, zf_ref, sf_ref, c_ref):
    b = b_ref[...]
    e = jnp.where(b > 0, b, jnp.exp(jnp.minimum(b, 0.0)) - 1.0)
    cz = jnp.dot(e, zf_ref[...], preferred_element_type=jnp.float32)
    cs = jnp.dot(e, sf_ref[...], preferred_element_type=jnp.float32)
    t = tm_ref[...]
    c_ref[...] = jnp.where(t == 2.0, cz, jnp.where(t == 3.0, cs, b))


def _child_finish(b, tm3, zft, sft):
    m = b.shape[0]
    return pl.pallas_call(
        _child_body,
        grid=(m // BM,),
        in_specs=[
            pl.BlockSpec((BM, HIDDEN), lambda i: (i, 0)),
            pl.BlockSpec((BM, 1), lambda i: (i, 0)),
            pl.BlockSpec((HIDDEN, HIDDEN), lambda i: (0, 0)),
            pl.BlockSpec((HIDDEN, HIDDEN), lambda i: (0, 0)),
        ],
        out_specs=pl.BlockSpec((BM, HIDDEN), lambda i: (i, 0)),
        out_shape=jax.ShapeDtypeStruct((m, HIDDEN), jnp.float32),
    )(b, tm3, zft, sft)


def _parent_body(s_ref, ta_ref, pf_ref, o_ref):
    s = s_ref[...]
    sp = jnp.dot(s, pf_ref[...], preferred_element_type=jnp.float32)
    e = jnp.where(s > 0, s, jnp.exp(jnp.minimum(s, 0.0)) - 1.0)
    t = ta_ref[...]
    o_ref[...] = jnp.where(t == 0.0, s, jnp.where(t == 1.0, sp, e))


def _parent_finish(s, ta3, pft):
    p = N_PARENTS
    return pl.pallas_call(
        _parent_body,
        grid=(p // BP,),
        in_specs=[
            pl.BlockSpec((BP, HIDDEN), lambda i: (i, 0)),
            pl.BlockSpec((BP, 1), lambda i: (i, 0)),
            pl.BlockSpec((HIDDEN, HIDDEN), lambda i: (0, 0)),
        ],
        out_specs=pl.BlockSpec((BP, HIDDEN), lambda i: (i, 0)),
        out_shape=jax.ShapeDtypeStruct((p, HIDDEN), jnp.float32),
    )(s, ta3, pft)


_SC_MESH = plsc.VectorSubcoreMesh(core_axis_name="c", subcore_axis_name="s")


@functools.partial(
    pl.kernel, mesh=_SC_MESH,
    out_type=jax.ShapeDtypeStruct((M_PAD, HIDDEN), jnp.float32),
    scratch_types=[
        pltpu.VMEM((4, CH_B), jnp.int32),
        pltpu.VMEM((CH_B, HIDDEN), jnp.float32),
        pltpu.VMEM((CH_B, HIDDEN), jnp.float32),
        pltpu.VMEM((CH_B, HIDDEN), jnp.float32),
        pltpu.VMEM((CH_B, HIDDEN), jnp.float32),
        pltpu.SemaphoreType.DMA,
        pltpu.SemaphoreType.DMA,
        pltpu.SemaphoreType.DMA,
    ],
)
def _sc_gather_sum(t_hbm, g_hbm, out_hbm, idx_v, b0, b1, b2, acc,
                   sem0, sem1, sem2):
    wid = lax.axis_index("s") * 2 + lax.axis_index("c")
    wbase = wid * ROWS_W

    def chunk(ci, _):
        base = wbase + ci * CH_B
        pltpu.sync_copy(g_hbm.at[pl.ds(base, CH_B)], idx_v.at[0])
        pltpu.sync_copy(g_hbm.at[pl.ds(M_PAD + base, CH_B)], idx_v.at[1])
        pltpu.sync_copy(g_hbm.at[pl.ds(2 * M_PAD + base, CH_B)], idx_v.at[2])
        c0 = pltpu.async_copy(t_hbm.at[idx_v.at[0]], b0, sem0)
        c1 = pltpu.async_copy(t_hbm.at[idx_v.at[1]], b1, sem1)
        c2 = pltpu.async_copy(t_hbm.at[idx_v.at[2]], b2, sem2)
        c0.wait()
        c1.wait()
        c2.wait()

        def row(r, _):
            for q in range(HIDDEN // 16):
                sl = pl.ds(q * 16, 16)
                acc[r, sl] = b0[r, sl] + b1[r, sl] + b2[r, sl]
            return _

        lax.fori_loop(0, CH_B, row, None)
        pltpu.sync_copy(acc, out_hbm.at[pl.ds(base, CH_B)])
        return _

    lax.fori_loop(0, NCH_B, chunk, None)


@functools.partial(
    pl.kernel, mesh=_SC_MESH,
    out_type=jax.ShapeDtypeStruct((P_PAD, HIDDEN), jnp.float32),
    scratch_types=[
        pltpu.VMEM((8, CH_D), jnp.int32),
        pltpu.VMEM((8, CH_D, HIDDEN), jnp.float32),
        pltpu.VMEM((CH_D, HIDDEN), jnp.float32),
        pltpu.SemaphoreType.DMA,
    ],
)
def _sc_segment_sum(c_hbm, gd_hbm, out_hbm, idx_v, bufs, acc, sem):
    wid = lax.axis_index("s") * 2 + lax.axis_index("c")
    wbase = wid * PAR_W

    def chunk(ci, _):
        base = wbase + ci * CH_D
        for j in range(8):
            pltpu.sync_copy(gd_hbm.at[pl.ds(j * P_PAD + base, CH_D)],
                            idx_v.at[j])
        copies = [
            pltpu.async_copy(c_hbm.at[idx_v.at[j]], bufs.at[j], sem)
            for j in range(8)
        ]
        for c in copies:
            c.wait()

        def row(r, _):
            for q in range(HIDDEN // 16):
                sl = pl.ds(q * 16, 16)
                v = bufs[0, r, sl]
                for j in range(1, 8):
                    v = v + bufs[j, r, sl]
                acc[r, sl] = v
            return _

        lax.fori_loop(0, CH_D, row, None)
        pltpu.sync_copy(acc, out_hbm.at[pl.ds(base, CH_D)])
        return _

    lax.fori_loop(0, NCH_D, chunk, None)


def kernel(x, p_w, p_final_w, z_w, z_final_w, s_w, s_final_w,
           initial_map, order_matrix, pooling, type_mask):
    n, h = x.shape
    k, m = order_matrix.shape
    p = N_PARENTS
    im = initial_map.astype(jnp.int32)
    om = order_matrix.astype(jnp.int32)
    tm = type_mask.astype(jnp.int32)
    pool = pooling.astype(jnp.int32)

    zspread = 8 * n + (jnp.arange(m, dtype=jnp.int32) % n)
    base0 = jnp.where(tm == 0, 0,
            jnp.where(tm == 1, n,
            jnp.where(tm == 2, 2 * n, 5 * n)))
    g0 = base0 + im

    def gk(kk):
        omk = om[kk]
        valid = (omk >= 0) & (tm >= 2)
        imk = im[jnp.clip(omk, 0, m - 1)]
        base = jnp.where(tm == 2, (2 + kk) * n, (5 + kk) * n)
        return jnp.where(valid, base + imk, zspread)

    pad = M_PAD - m
    zpad = 8 * n + (jnp.arange(pad, dtype=jnp.int32) % n)
    g0 = jnp.concatenate([g0, zpad])
    g1 = jnp.concatenate([gk(1), zpad])
    g2 = jnp.concatenate([gk(2), zpad])
    tm_pad = jnp.concatenate([tm, jnp.zeros((pad,), jnp.int32)])
    tm3 = tm_pad.astype(jnp.float32).reshape(M_PAD, 1)

    starts = jnp.searchsorted(pool, jnp.arange(p)).astype(jnp.int32)
    tm_after = tm[starts]
    cnt = jnp.where(tm_after == 0, 1, 8)
    jj = jnp.arange(8, dtype=jnp.int32)[:, None]
    zsd = m + ((jj * p + jnp.arange(p, dtype=jnp.int32)[None, :]) % pad)
    gd = jnp.where(jj < cnt[None, :], starts[None, :] + jj, zsd)
    zsd_tail = m + (jnp.arange(8 * (P_PAD - p), dtype=jnp.int32)
                    .reshape(8, P_PAD - p) % pad)
    gd = jnp.concatenate([gd, zsd_tail], axis=1)
    gd = gd.reshape(8 * P_PAD)
    ta3 = tm_after.astype(jnp.float32).reshape(p, 1)

    wcat = jnp.concatenate([p_w.T] + [z_w[i].T for i in range(3)]
                           + [s_w[i].T for i in range(3)], axis=1)
    t_tab = _expand(x, wcat).reshape(9 * n, h)

    g = jnp.concatenate([g0, g1, g2])
    b = _sc_gather_sum(t_tab, g)

    c = _child_finish(b, tm3, z_final_w.T, s_final_w.T)

    s = _sc_segment_sum(c, gd)

    return _parent_finish(s, ta3, p_final_w.T)


# speedup vs baseline: 7.0437x; 1.9674x over previous
import functools

import jax
import jax.numpy as jnp
from jax import lax
from jax.experimental import pallas as pl
from jax.experimental.pallas import tpu as pltpu
from jax.experimental.pallas import tpu_sc as plsc

N_NODES = 10000
HIDDEN = 128
N_PARENTS = 20000

BN = 1000
BM = 1024
BP = 2000
M_PAD = 126976
NWORK = 32
ROWS_W = M_PAD // NWORK
CH_B = 128
NCH_B = ROWS_W // CH_B
P_PAD = 20480
PAR_W = P_PAD // NWORK
CH_D = 64
NCH_D = PAR_W // CH_D


def _expand_body(x_ref, w_ref, t_ref):
    xb = x_ref[...]
    y = jnp.dot(xb, w_ref[...], preferred_element_type=jnp.float32)
    t_ref[0] = xb
    for j in range(7):
        t_ref[1 + j] = y[:, HIDDEN * j:HIDDEN * (j + 1)]
    t_ref[8] = jnp.zeros_like(xb)


def _expand(x, wcat):
    n = x.shape[0]
    return pl.pallas_call(
        _expand_body,
        grid=(n // BN,),
        in_specs=[
            pl.BlockSpec((BN, HIDDEN), lambda i: (i, 0)),
            pl.BlockSpec((HIDDEN, 7 * HIDDEN), lambda i: (0, 0)),
        ],
        out_specs=pl.BlockSpec((9, BN, HIDDEN), lambda i: (0, i, 0)),
        out_shape=jax.ShapeDtypeStruct((9, n, HIDDEN), jnp.float32),
    )(x, wcat)


def _child_body(b_ref, tm_ref, zf_ref, sf_ref, c_ref):
    b = b_ref[...]
    e = jnp.where(b > 0, b, jnp.exp(jnp.minimum(b, 0.0)) - 1.0)
    cz = jnp.dot(e, zf_ref[...], preferred_element_type=jnp.float32)
    cs = jnp.dot(e, sf_ref[...], preferred_element_type=jnp.float32)
    t = tm_ref[...]
    c_ref[...] = jnp.where(t == 2.0, cz, jnp.where(t == 3.0, cs, b))


def _child_finish(b, tm3, zft, sft):
    m = b.shape[0]
    return pl.pallas_call(
        _child_body,
        grid=(m // BM,),
        in_specs=[
            pl.BlockSpec((BM, HIDDEN), lambda i: (i, 0)),
            pl.BlockSpec((BM, 1), lambda i: (i, 0)),
            pl.BlockSpec((HIDDEN, HIDDEN), lambda i: (0, 0)),
            pl.BlockSpec((HIDDEN, HIDDEN), lambda i: (0, 0)),
        ],
        out_specs=pl.BlockSpec((BM, HIDDEN), lambda i: (i, 0)),
        out_shape=jax.ShapeDtypeStruct((m, HIDDEN), jnp.float32),
    )(b, tm3, zft, sft)


def _parent_body(s_ref, ta_ref, pf_ref, o_ref):
    s = s_ref[...]
    sp = jnp.dot(s, pf_ref[...], preferred_element_type=jnp.float32)
    e = jnp.where(s > 0, s, jnp.exp(jnp.minimum(s, 0.0)) - 1.0)
    t = ta_ref[...]
    o_ref[...] = jnp.where(t == 0.0, s, jnp.where(t == 1.0, sp, e))


def _parent_finish(s, ta3, pft):
    p = N_PARENTS
    return pl.pallas_call(
        _parent_body,
        grid=(p // BP,),
        in_specs=[
            pl.BlockSpec((BP, HIDDEN), lambda i: (i, 0)),
            pl.BlockSpec((BP, 1), lambda i: (i, 0)),
            pl.BlockSpec((HIDDEN, HIDDEN), lambda i: (0, 0)),
        ],
        out_specs=pl.BlockSpec((BP, HIDDEN), lambda i: (i, 0)),
        out_shape=jax.ShapeDtypeStruct((p, HIDDEN), jnp.float32),
    )(s, ta3, pft)


_SC_MESH = plsc.VectorSubcoreMesh(core_axis_name="c", subcore_axis_name="s")

CB = 64
NCB = ROWS_W // CB
NGRP = ROWS_W // 16


@functools.partial(
    pl.kernel, mesh=_SC_MESH,
    out_type=jax.ShapeDtypeStruct((M_PAD, HIDDEN), jnp.float32),
    scratch_types=[
        pltpu.VMEM((6, ROWS_W), jnp.int32),
        pltpu.VMEM((2, ROWS_W), jnp.int32),
        pltpu.VMEM((3, ROWS_W), jnp.int32),
        pltpu.VMEM((2, 3, CB, HIDDEN), jnp.float32),
        pltpu.VMEM((CB, HIDDEN), jnp.float32),
        pltpu.SemaphoreType.DMA,
        pltpu.SemaphoreType.DMA,
        pltpu.SemaphoreType.DMA,
    ],
)
def _sc_gather_sum(t_hbm, im_h, o1r_h, o1c_h, o2r_h, o2c_h, tm_h,
                   out_hbm, intv, imgv, gv, rb, acc, sg0, sg1, selem):
    n = N_NODES
    wid = lax.axis_index("s") * 2 + lax.axis_index("c")
    wbase = wid * ROWS_W

    sl_w = pl.ds(wbase, ROWS_W)
    pltpu.sync_copy(im_h.at[sl_w], intv.at[0])
    pltpu.sync_copy(o1r_h.at[sl_w], intv.at[1])
    pltpu.sync_copy(o1c_h.at[sl_w], intv.at[2])
    pltpu.sync_copy(o2r_h.at[sl_w], intv.at[3])
    pltpu.sync_copy(o2c_h.at[sl_w], intv.at[4])
    pltpu.sync_copy(tm_h.at[sl_w], intv.at[5])

    handles = []
    for c in range(ROWS_W // 128):
        slc = pl.ds(c * 128, 128)
        handles.append(pltpu.async_copy(
            im_h.at[intv.at[2, slc]], imgv.at[0, slc], selem))
        handles.append(pltpu.async_copy(
            im_h.at[intv.at[4, slc]], imgv.at[1, slc], selem))
    for hdl in handles:
        hdl.wait()

    lanes = lax.iota(jnp.int32, 16)

    def grp(q, _):
        s16 = pl.ds(q * 16, 16)
        t = intv[5, s16]
        imv = intv[0, s16]
        gi = wbase + q * 16 + lanes
        zsp = 8 * n + lax.rem(gi, n)
        b0 = jnp.where(t == 0, 0,
             jnp.where(t == 1, n,
             jnp.where(t == 2, 2 * n, 5 * n)))
        gv[0, s16] = b0 + imv
        for slot, (raw_row, img_row) in enumerate(((1, 0), (3, 1))):
            o = intv[raw_row, s16]
            imk = imgv[img_row, s16]
            ok = (o >= 0) & (t >= 2)
            bk = jnp.where(t == 2, (3 + slot) * n, (6 + slot) * n)
            gv[1 + slot, s16] = jnp.where(ok, bk + imk, zsp)
        return _

    lax.fori_loop(0, NGRP, grp, None)

    def fire(s, sem, coff):
        for k in range(3):
            pltpu.async_copy(t_hbm.at[gv.at[k, pl.ds(coff, CB)]],
                             rb.at[s, k], sem)

    def drain(s, sem, coff):
        for k in range(3):
            pltpu.make_async_copy(t_hbm.at[gv.at[k, pl.ds(coff, CB)]],
                                  rb.at[s, k], sem).wait()

    def sum_out(s, coff):
        def row(r, _):
            for q in range(HIDDEN // 16):
                sq = pl.ds(q * 16, 16)
                acc[r, sq] = (rb[s, 0, r, sq] + rb[s, 1, r, sq]
                              + rb[s, 2, r, sq])
            return _

        lax.fori_loop(0, CB, row, None)
        pltpu.sync_copy(acc, out_hbm.at[pl.ds(wbase + coff, CB)])

    fire(0, sg0, 0)

    def pair(i, _):
        c0 = 2 * i * CB
        fire(1, sg1, c0 + CB)
        drain(0, sg0, c0)
        sum_out(0, c0)

        @pl.when(i < NCB // 2 - 1)
        def _fire_next():
            fire(0, sg0, c0 + 2 * CB)

        drain(1, sg1, c0 + CB)
        sum_out(1, c0 + CB)
        return _

    lax.fori_loop(0, NCB // 2, pair, None)


PD = 16
NPD = PAR_W // PD


@functools.partial(
    pl.kernel, mesh=_SC_MESH,
    out_type=jax.ShapeDtypeStruct((P_PAD, HIDDEN), jnp.float32),
    scratch_types=[
        pltpu.VMEM((PAR_W * 8,), jnp.int32),
        pltpu.VMEM((2, PD * 8, HIDDEN), jnp.float32),
        pltpu.VMEM((PD, HIDDEN), jnp.float32),
        pltpu.SemaphoreType.DMA,
        pltpu.SemaphoreType.DMA,
    ],
)
def _sc_segment_sum(c_hbm, gd_hbm, out_hbm, gdv, rb, acc, sg0, sg1):
    wid = lax.axis_index("s") * 2 + lax.axis_index("c")
    wbase = wid * PAR_W
    pltpu.sync_copy(gd_hbm.at[pl.ds(wbase * 8, PAR_W * 8)], gdv)

    def fire(s, sem, coff):
        pltpu.async_copy(c_hbm.at[gdv.at[pl.ds(coff * 8, PD * 8)]],
                         rb.at[s], sem)

    def drain(s, sem, coff):
        pltpu.make_async_copy(c_hbm.at[gdv.at[pl.ds(coff * 8, PD * 8)]],
                              rb.at[s], sem).wait()

    def sum_out(s, coff):
        def par(r, _):
            for q in range(HIDDEN // 16):
                sq = pl.ds(q * 16, 16)
                v = rb[s, r * 8, sq]
                for j in range(1, 8):
                    v = v + rb[s, r * 8 + j, sq]
                acc[r, sq] = v
            return _

        lax.fori_loop(0, PD, par, None)
        pltpu.sync_copy(acc, out_hbm.at[pl.ds(wbase + coff, PD)])

    fire(0, sg0, 0)

    def pair(i, _):
        c0 = 2 * i * PD
        fire(1, sg1, c0 + PD)
        drain(0, sg0, c0)
        sum_out(0, c0)

        @pl.when(i < NPD // 2 - 1)
        def _fire_next():
            fire(0, sg0, c0 + 2 * PD)

        drain(1, sg1, c0 + PD)
        sum_out(1, c0 + PD)
        return _

    lax.fori_loop(0, NPD // 2, pair, None)


def kernel(x, p_w, p_final_w, z_w, z_final_w, s_w, s_final_w,
           initial_map, order_matrix, pooling, type_mask):
    n, h = x.shape
    k, m = order_matrix.shape
    p = N_PARENTS
    im = initial_map.astype(jnp.int32)
    om = order_matrix.astype(jnp.int32)
    tm = type_mask.astype(jnp.int32)
    pool = pooling.astype(jnp.int32)

    pad = M_PAD - m
    izpad = jnp.zeros((pad,), jnp.int32)
    im_p = jnp.concatenate(
        [im, 8 * n + (jnp.arange(pad, dtype=jnp.int32) % n)])
    o1r = jnp.concatenate([om[1], jnp.full((pad,), -1, jnp.int32)])
    o2r = jnp.concatenate([om[2], jnp.full((pad,), -1, jnp.int32)])
    o1c = jnp.maximum(o1r, 0)
    o2c = jnp.maximum(o2r, 0)
    tm_pad = jnp.concatenate([tm, izpad])
    tm3 = tm_pad.astype(jnp.float32).reshape(M_PAD, 1)

    mi = jnp.arange(m, dtype=jnp.int32)
    is_start = jnp.concatenate(
        [jnp.ones((1,), bool), pool[1:] != pool[:-1]])
    seg_start = lax.cummax(jnp.where(is_start, mi, -1), axis=0)
    slot = mi - seg_start
    gd = m + (jnp.arange(8 * P_PAD, dtype=jnp.int32) % pad)
    gd = gd.at[pool * 8 + slot].set(mi)
    tm_after = jnp.zeros((p,), jnp.int32).at[pool].set(tm)
    ta3 = tm_after.astype(jnp.float32).reshape(p, 1)

    wcat = jnp.concatenate([p_w.T] + [z_w[i].T for i in range(3)]
                           + [s_w[i].T for i in range(3)], axis=1)
    t_tab = _expand(x, wcat).reshape(9 * n, h)

    b = _sc_gather_sum(t_tab, im_p, o1r, o1c, o2r, o2c, tm_pad)

    c = _child_finish(b, tm3, z_final_w.T, s_final_w.T)

    s = _sc_segment_sum(c, gd)

    return _parent_finish(s, ta3, p_final_w.T)


# speedup vs baseline: 7.0896x; 1.0065x over previous
import functools

import jax
import jax.numpy as jnp
from jax import lax
from jax.experimental import pallas as pl
from jax.experimental.pallas import tpu as pltpu
from jax.experimental.pallas import tpu_sc as plsc

N_NODES = 10000
HIDDEN = 128
N_PARENTS = 20000

BN = 1000
BM = 1024
BP = 2000
M_PAD = 126976
NWORK = 32
ROWS_W = M_PAD // NWORK
CH_B = 128
NCH_B = ROWS_W // CH_B
P_PAD = 20480
PAR_W = P_PAD // NWORK
CH_D = 64
NCH_D = PAR_W // CH_D


def _expand_body(x_ref, w_ref, t_ref):
    xb = x_ref[...]
    y = jnp.dot(xb, w_ref[...], preferred_element_type=jnp.float32)
    t_ref[0] = xb
    for j in range(7):
        t_ref[1 + j] = y[:, HIDDEN * j:HIDDEN * (j + 1)]
    t_ref[8] = jnp.zeros_like(xb)


def _expand(x, wcat):
    n = x.shape[0]
    return pl.pallas_call(
        _expand_body,
        grid=(n // BN,),
        in_specs=[
            pl.BlockSpec((BN, HIDDEN), lambda i: (i, 0)),
            pl.BlockSpec((HIDDEN, 7 * HIDDEN), lambda i: (0, 0)),
        ],
        out_specs=pl.BlockSpec((9, BN, HIDDEN), lambda i: (0, i, 0)),
        out_shape=jax.ShapeDtypeStruct((9, n, HIDDEN), jnp.float32),
    )(x, wcat)


def _child_body(b_ref, tm_ref, zf_ref, sf_ref, c_ref):
    b = b_ref[...]
    e = jnp.where(b > 0, b, jnp.exp(jnp.minimum(b, 0.0)) - 1.0)
    cz = jnp.dot(e, zf_ref[...], preferred_element_type=jnp.float32)
    cs = jnp.dot(e, sf_ref[...], preferred_element_type=jnp.float32)
    t = tm_ref[...]
    c_ref[...] = jnp.where(t == 2.0, cz, jnp.where(t == 3.0, cs, b))


def _child_finish(b, tm3, zft, sft):
    m = b.shape[0]
    return pl.pallas_call(
        _child_body,
        grid=(m // BM,),
        in_specs=[
            pl.BlockSpec((BM, HIDDEN), lambda i: (i, 0)),
            pl.BlockSpec((BM, 1), lambda i: (i, 0)),
            pl.BlockSpec((HIDDEN, HIDDEN), lambda i: (0, 0)),
            pl.BlockSpec((HIDDEN, HIDDEN), lambda i: (0, 0)),
        ],
        out_specs=pl.BlockSpec((BM, HIDDEN), lambda i: (i, 0)),
        out_shape=jax.ShapeDtypeStruct((m, HIDDEN), jnp.float32),
    )(b, tm3, zft, sft)


def _parent_body(s_ref, ta_ref, pf_ref, o_ref):
    s = s_ref[...]
    sp = jnp.dot(s, pf_ref[...], preferred_element_type=jnp.float32)
    e = jnp.where(s > 0, s, jnp.exp(jnp.minimum(s, 0.0)) - 1.0)
    t = ta_ref[...]
    o_ref[...] = jnp.where(t == 0.0, s, jnp.where(t == 1.0, sp, e))


def _parent_finish(s, ta3, pft):
    p = N_PARENTS
    return pl.pallas_call(
        _parent_body,
        grid=(p // BP,),
        in_specs=[
            pl.BlockSpec((BP, HIDDEN), lambda i: (i, 0)),
            pl.BlockSpec((BP, 1), lambda i: (i, 0)),
            pl.BlockSpec((HIDDEN, HIDDEN), lambda i: (0, 0)),
        ],
        out_specs=pl.BlockSpec((BP, HIDDEN), lambda i: (i, 0)),
        out_shape=jax.ShapeDtypeStruct((p, HIDDEN), jnp.float32),
    )(s, ta3, pft)


_SC_MESH = plsc.VectorSubcoreMesh(core_axis_name="c", subcore_axis_name="s")

CB = 64
NCB = ROWS_W // CB
NGRP = ROWS_W // 16


@functools.partial(
    pl.kernel, mesh=_SC_MESH,
    out_type=jax.ShapeDtypeStruct((M_PAD, HIDDEN), jnp.float32),
    scratch_types=[
        pltpu.VMEM((6, ROWS_W), jnp.int32),
        pltpu.VMEM((2, 2, CB), jnp.int32),
        pltpu.VMEM((2, 3, CB), jnp.int32),
        pltpu.VMEM((2, 3, CB, HIDDEN), jnp.float32),
        pltpu.VMEM((CB, HIDDEN), jnp.float32),
        pltpu.SemaphoreType.DMA,
        pltpu.SemaphoreType.DMA,
        pltpu.SemaphoreType.DMA,
        pltpu.SemaphoreType.DMA,
    ],
)
def _sc_gather_sum(t_hbm, im_h, o1r_h, o1c_h, o2r_h, o2c_h, tm_h,
                   out_hbm, intv, imgv, gv, rb, acc, sg0, sg1, se0, se1):
    n = N_NODES
    wid = lax.axis_index("s") * 2 + lax.axis_index("c")
    wbase = wid * ROWS_W

    sl_w = pl.ds(wbase, ROWS_W)
    pltpu.sync_copy(im_h.at[sl_w], intv.at[0])
    pltpu.sync_copy(o1r_h.at[sl_w], intv.at[1])
    pltpu.sync_copy(o1c_h.at[sl_w], intv.at[2])
    pltpu.sync_copy(o2r_h.at[sl_w], intv.at[3])
    pltpu.sync_copy(o2c_h.at[sl_w], intv.at[4])
    pltpu.sync_copy(tm_h.at[sl_w], intv.at[5])

    lanes = lax.iota(jnp.int32, 16)
    esems = (se0, se1)
    gsems = (sg0, sg1)

    def fire_elem(s, coff):
        pltpu.async_copy(im_h.at[intv.at[2, pl.ds(coff, CB)]],
                         imgv.at[s, 0], esems[s])
        pltpu.async_copy(im_h.at[intv.at[4, pl.ds(coff, CB)]],
                         imgv.at[s, 1], esems[s])

    def drain_elem(s, coff):
        pltpu.make_async_copy(im_h.at[intv.at[2, pl.ds(coff, CB)]],
                              imgv.at[s, 0], esems[s]).wait()
        pltpu.make_async_copy(im_h.at[intv.at[4, pl.ds(coff, CB)]],
                              imgv.at[s, 1], esems[s]).wait()

    def compose(s, coff):
        def grp(q, _):
            s16 = pl.ds(coff + q * 16, 16)
            d16 = pl.ds(q * 16, 16)
            t = intv[5, s16]
            imv = intv[0, s16]
            gi = wbase + coff + q * 16 + lanes
            zsp = 8 * n + lax.rem(gi, n)
            b0 = jnp.where(t == 0, 0,
                 jnp.where(t == 1, n,
                 jnp.where(t == 2, 2 * n, 5 * n)))
            gv[s, 0, d16] = b0 + imv
            for slot, raw_row in enumerate((1, 3)):
                o = intv[raw_row, s16]
                imk = imgv[s, slot, d16]
                ok = (o >= 0) & (t >= 2)
                bk = jnp.where(t == 2, (3 + slot) * n, (6 + slot) * n)
                gv[s, 1 + slot, d16] = jnp.where(ok, bk + imk, zsp)
            return _

        lax.fori_loop(0, CB // 16, grp, None)

    def fire_rows(s):
        for k in range(3):
            pltpu.async_copy(t_hbm.at[gv.at[s, k]], rb.at[s, k], gsems[s])

    def drain_rows(s):
        for k in range(3):
            pltpu.make_async_copy(t_hbm.at[gv.at[s, k]], rb.at[s, k],
                                  gsems[s]).wait()

    def sum_out(s, coff):
        def row(r, _):
            for q in range(HIDDEN // 16):
                sq = pl.ds(q * 16, 16)
                acc[r, sq] = (rb[s, 0, r, sq] + rb[s, 1, r, sq]
                              + rb[s, 2, r, sq])
            return _

        lax.fori_loop(0, CB, row, None)
        pltpu.sync_copy(acc, out_hbm.at[pl.ds(wbase + coff, CB)])

    fire_elem(0, 0)
    drain_elem(0, 0)
    compose(0, 0)
    fire_rows(0)
    fire_elem(1, CB)

    def pair(i, _):
        c0 = 2 * i * CB
        drain_elem(1, c0 + CB)
        compose(1, c0 + CB)
        fire_rows(1)

        @pl.when(i < NCB // 2 - 1)
        def _e0():
            fire_elem(0, c0 + 2 * CB)

        drain_rows(0)
        sum_out(0, c0)

        @pl.when(i < NCB // 2 - 1)
        def _c0():
            drain_elem(0, c0 + 2 * CB)
            compose(0, c0 + 2 * CB)

        drain_rows(1)

        @pl.when(i < NCB // 2 - 1)
        def _r0():
            fire_rows(0)
            fire_elem(1, c0 + 3 * CB)

        sum_out(1, c0 + CB)
        return _

    lax.fori_loop(0, NCB // 2, pair, None)


PD = 16
NPD = PAR_W // PD


@functools.partial(
    pl.kernel, mesh=_SC_MESH,
    out_type=jax.ShapeDtypeStruct((P_PAD, HIDDEN), jnp.float32),
    scratch_types=[
        pltpu.VMEM((PAR_W * 8,), jnp.int32),
        pltpu.VMEM((2, PD * 8, HIDDEN), jnp.float32),
        pltpu.VMEM((PD, HIDDEN), jnp.float32),
        pltpu.SemaphoreType.DMA,
        pltpu.SemaphoreType.DMA,
    ],
)
def _sc_segment_sum(c_hbm, gd_hbm, out_hbm, gdv, rb, acc, sg0, sg1):
    wid = lax.axis_index("s") * 2 + lax.axis_index("c")
    wbase = wid * PAR_W
    pltpu.sync_copy(gd_hbm.at[pl.ds(wbase * 8, PAR_W * 8)], gdv)

    def fire(s, sem, coff):
        pltpu.async_copy(c_hbm.at[gdv.at[pl.ds(coff * 8, PD * 8)]],
                         rb.at[s], sem)

    def drain(s, sem, coff):
        pltpu.make_async_copy(c_hbm.at[gdv.at[pl.ds(coff * 8, PD * 8)]],
                              rb.at[s], sem).wait()

    def sum_out(s, coff):
        def par(r, _):
            for q in range(HIDDEN // 16):
                sq = pl.ds(q * 16, 16)
                v = rb[s, r * 8, sq]
                for j in range(1, 8):
                    v = v + rb[s, r * 8 + j, sq]
                acc[r, sq] = v
            return _

        lax.fori_loop(0, PD, par, None)
        pltpu.sync_copy(acc, out_hbm.at[pl.ds(wbase + coff, PD)])

    fire(0, sg0, 0)

    def pair(i, _):
        c0 = 2 * i * PD
        fire(1, sg1, c0 + PD)
        drain(0, sg0, c0)
        sum_out(0, c0)

        @pl.when(i < NPD // 2 - 1)
        def _fire_next():
            fire(0, sg0, c0 + 2 * PD)

        drain(1, sg1, c0 + PD)
        sum_out(1, c0 + PD)
        return _

    lax.fori_loop(0, NPD // 2, pair, None)


def kernel(x, p_w, p_final_w, z_w, z_final_w, s_w, s_final_w,
           initial_map, order_matrix, pooling, type_mask):
    n, h = x.shape
    k, m = order_matrix.shape
    p = N_PARENTS
    im = initial_map.astype(jnp.int32)
    om = order_matrix.astype(jnp.int32)
    tm = type_mask.astype(jnp.int32)
    pool = pooling.astype(jnp.int32)

    pad = M_PAD - m
    izpad = jnp.zeros((pad,), jnp.int32)
    im_p = jnp.concatenate(
        [im, 8 * n + (jnp.arange(pad, dtype=jnp.int32) % n)])
    o1r = jnp.concatenate([om[1], jnp.full((pad,), -1, jnp.int32)])
    o2r = jnp.concatenate([om[2], jnp.full((pad,), -1, jnp.int32)])
    o1c = jnp.maximum(o1r, 0)
    o2c = jnp.maximum(o2r, 0)
    tm_pad = jnp.concatenate([tm, izpad])
    tm3 = tm_pad.astype(jnp.float32).reshape(M_PAD, 1)

    mi = jnp.arange(m, dtype=jnp.int32)
    is_start = jnp.concatenate(
        [jnp.ones((1,), bool), pool[1:] != pool[:-1]])
    seg_start = lax.cummax(jnp.where(is_start, mi, -1), axis=0)
    slot = mi - seg_start
    gd = m + (jnp.arange(8 * P_PAD, dtype=jnp.int32) % pad)
    gd = gd.at[pool * 8 + slot].set(mi)
    tm_after = jnp.zeros((p,), jnp.int32).at[pool].set(tm)
    ta3 = tm_after.astype(jnp.float32).reshape(p, 1)

    wcat = jnp.concatenate([p_w.T] + [z_w[i].T for i in range(3)]
                           + [s_w[i].T for i in range(3)], axis=1)
    t_tab = _expand(x, wcat).reshape(9 * n, h)

    b = _sc_gather_sum(t_tab, im_p, o1r, o1c, o2r, o2c, tm_pad)

    c = _child_finish(b, tm3, z_final_w.T, s_final_w.T)

    s = _sc_segment_sum(c, gd)

    return _parent_finish(s, ta3, p_final_w.T)
